# Initial kernel scaffold; baseline (speedup 1.0000x reference)
import jax
import jax.numpy as jnp
from jax.experimental import pallas as pl


def kernel(x, edge_index, g0_W1, g0_b1, g0_W2, g0_b2, g1_W1, g1_b1, g1_W2, g1_b2, out_W, out_b, ep_W, ep_b):
    raise NotImplementedError("write your pallas kernel here")


# speedup vs baseline: 11.0140x; 11.0140x over previous
import functools

import jax
import jax.numpy as jnp
from jax import lax
from jax.experimental import pallas as pl
from jax.experimental.pallas import tpu as pltpu
from jax.experimental.pallas import tpu_sc as plsc

N = 1024
E = 32768
D = 128
NC = 2
NS = 16
NW = NC * NS
EDGES_PER_W = E // NW
CHUNK = 256
NCHUNK = EDGES_PER_W // CHUNK
ROWS_PER_SUB = N // NS


def _seg_sum_body(h_hbm, src_hbm, dst_hbm, zero_hbm, out_hbm,
                  src_v, dst_v, rows_v, acc_sh, sem):
    c = lax.axis_index("c")
    s = lax.axis_index("s")
    wid = c * NS + s
    pltpu.sync_copy(zero_hbm.at[pl.ds(s * ROWS_PER_SUB, ROWS_PER_SUB)],
                    acc_sh.at[pl.ds(s * ROWS_PER_SUB, ROWS_PER_SUB)])
    plsc.subcore_barrier()
    base = wid * EDGES_PER_W
    for j in range(NCHUNK):
        off = base + j * CHUNK
        pltpu.sync_copy(src_hbm.at[pl.ds(off, CHUNK)], src_v)
        pltpu.sync_copy(dst_hbm.at[pl.ds(off, CHUNK)], dst_v)
        pltpu.async_copy(h_hbm.at[src_v], rows_v, sem).wait()
        pltpu.sync_copy(rows_v, acc_sh.at[dst_v], add=True)
    plsc.subcore_barrier()
    pltpu.sync_copy(acc_sh.at[pl.ds(s * ROWS_PER_SUB, ROWS_PER_SUB)],
                    out_hbm.at[c, pl.ds(s * ROWS_PER_SUB, ROWS_PER_SUB)])


@functools.lru_cache(maxsize=None)
def _make_seg_sum():
    return pl.kernel(
        _seg_sum_body,
        out_type=jax.ShapeDtypeStruct((NC, N, D), jnp.float32),
        mesh=plsc.VectorSubcoreMesh(core_axis_name="c", subcore_axis_name="s",
                                    num_cores=NC, num_subcores=NS),
        scratch_types=[
            pltpu.VMEM((CHUNK,), jnp.int32),
            pltpu.VMEM((CHUNK,), jnp.int32),
            pltpu.VMEM((CHUNK, D), jnp.float32),
            pltpu.VMEM_SHARED((N, D), jnp.float32),
            pltpu.SemaphoreType.DMA,
        ],
    )


def _mlp_body(x_ref, p_ref, w1_ref, b1_ref, w2_ref, b2_ref, o_ref):
    z = x_ref[...] + p_ref[0] + p_ref[1]
    t = jnp.dot(z, w1_ref[...], preferred_element_type=jnp.float32) + b1_ref[...]
    t = jnp.maximum(t, 0.0)
    h = jnp.dot(t, w2_ref[...], preferred_element_type=jnp.float32) + b2_ref[...]
    o_ref[...] = jnp.maximum(h, 0.0)


_mlp = pl.pallas_call(
    _mlp_body,
    out_shape=jax.ShapeDtypeStruct((N, D), jnp.float32),
)


def _head_body(h_ref, p_ref, w1_ref, b1_ref, w2_ref, b2_ref,
               ow_ref, ob_ref, epw1_ref, epw2t_ref, epb_ref, o_ref):
    z = h_ref[...] + p_ref[0] + p_ref[1]
    t = jnp.dot(z, w1_ref[...], preferred_element_type=jnp.float32) + b1_ref[...]
    t = jnp.maximum(t, 0.0)
    h2 = jnp.dot(t, w2_ref[...], preferred_element_type=jnp.float32) + b2_ref[...]
    h2 = jnp.maximum(h2, 0.0)
    hh = jnp.dot(h2, ow_ref[...], preferred_element_type=jnp.float32) + ob_ref[...]
    a = jnp.dot(hh, epw1_ref[...], preferred_element_type=jnp.float32)
    brow = lax.dot_general(epw2t_ref[...], hh, (((1,), (1,)), ((), ())),
                           preferred_element_type=jnp.float32)
    o_ref[...] = a + brow + epb_ref[...]


_head = pl.pallas_call(
    _head_body,
    out_shape=jax.ShapeDtypeStruct((N, N), jnp.float32),
)


def kernel(x, edge_index, g0_W1, g0_b1, g0_W2, g0_b2, g1_W1, g1_b1,
           g1_W2, g1_b2, out_W, out_b, ep_W, ep_b):
    src = edge_index[0]
    dst = edge_index[1]
    zeros = jnp.zeros((N, D), jnp.float32)

    seg_sum = _make_seg_sum()
    p0 = seg_sum(x, src, dst, zeros)
    h1 = _mlp(x, p0, g0_W1, g0_b1.reshape(1, D), g0_W2, g0_b2.reshape(1, D))
    p1 = seg_sum(h1, src, dst, zeros)
    out = _head(h1, p1, g1_W1, g1_b1.reshape(1, D), g1_W2, g1_b2.reshape(1, D),
                out_W, out_b.reshape(1, D), ep_W[:D, :],
                ep_W[D:, :].reshape(1, D), ep_b.reshape(1, 1))
    return out


# speedup vs baseline: 13.7323x; 1.2468x over previous
import functools

import jax
import jax.numpy as jnp
from jax import lax
from jax.experimental import pallas as pl
from jax.experimental.pallas import tpu as pltpu
from jax.experimental.pallas import tpu_sc as plsc

N = 1024
E = 32768
D = 128
NC = 2
NS = 16
NW = NC * NS
EPT = E // NW
NCH = EPT // 16
NB = EPT // 128
ZW = (N * N) // NS


def _abuild_body(edge_hbm, zero_hbm, ones_hbm, out_hbm,
                 srcdst_v, idx_v, ones_v, acc_sh, sem):
    c = lax.axis_index("c")
    s = lax.axis_index("s")
    pltpu.sync_copy(zero_hbm.at[pl.ds(s * ZW, ZW)], acc_sh.at[pl.ds(s * ZW, ZW)])
    pltpu.sync_copy(ones_hbm, ones_v)
    base = (c * NS + s) * EPT
    pltpu.sync_copy(edge_hbm.at[0, pl.ds(base, EPT)], srcdst_v.at[0])
    pltpu.sync_copy(edge_hbm.at[1, pl.ds(base, EPT)], srcdst_v.at[1])
    for i in range(NCH):
        sv = srcdst_v[0, pl.ds(i * 16, 16)]
        dv = srcdst_v[1, pl.ds(i * 16, 16)]
        idx_v[i // 8, pl.ds((i % 8) * 16, 16)] = dv * N + sv
    plsc.subcore_barrier()
    copies = [pltpu.async_copy(ones_v, acc_sh.at[idx_v.at[b]], sem, add=True)
              for b in range(NB)]
    for cp in copies:
        cp.wait()
    plsc.subcore_barrier()
    pltpu.sync_copy(acc_sh.at[pl.ds(s * ZW, ZW)], out_hbm.at[c, pl.ds(s * ZW, ZW)])


@functools.lru_cache(maxsize=None)
def _make_abuild():
    return pl.kernel(
        _abuild_body,
        out_type=jax.ShapeDtypeStruct((NC, N * N), jnp.float32),
        mesh=plsc.VectorSubcoreMesh(core_axis_name="c", subcore_axis_name="s",
                                    num_cores=NC, num_subcores=NS),
        scratch_types=[
            pltpu.VMEM((2, EPT), jnp.int32),
            pltpu.VMEM((NB, 128), jnp.int32),
            pltpu.VMEM((128,), jnp.float32),
            pltpu.VMEM_SHARED((N * N,), jnp.float32),
            pltpu.SemaphoreType.DMA,
        ],
    )


def _dense_body(a_ref, x_ref, w01_ref, b01_ref, w02_ref, b02_ref,
                w11_ref, b11_ref, w12_ref, b12_ref,
                ow_ref, ob_ref, epw1_ref, epw2t_ref, epb_ref, o_ref):
    A = a_ref[0] + a_ref[1]
    x = x_ref[...]
    z = x + jnp.dot(A, x, preferred_element_type=jnp.float32, precision=lax.Precision.HIGHEST)
    t = jnp.maximum(jnp.dot(z, w01_ref[...], preferred_element_type=jnp.float32)
                    + b01_ref[...], 0.0)
    h1 = jnp.maximum(jnp.dot(t, w02_ref[...], preferred_element_type=jnp.float32)
                     + b02_ref[...], 0.0)
    z2 = h1 + jnp.dot(A, h1, preferred_element_type=jnp.float32, precision=lax.Precision.HIGHEST)
    t2 = jnp.maximum(jnp.dot(z2, w11_ref[...], preferred_element_type=jnp.float32)
                     + b11_ref[...], 0.0)
    h2 = jnp.maximum(jnp.dot(t2, w12_ref[...], preferred_element_type=jnp.float32)
                     + b12_ref[...], 0.0)
    hh = jnp.dot(h2, ow_ref[...], preferred_element_type=jnp.float32) + ob_ref[...]
    a = jnp.dot(hh, epw1_ref[...], preferred_element_type=jnp.float32)
    brow = lax.dot_general(epw2t_ref[...], hh, (((1,), (1,)), ((), ())),
                           preferred_element_type=jnp.float32)
    o_ref[...] = a + brow + epb_ref[...]


_dense = pl.pallas_call(
    _dense_body,
    out_shape=jax.ShapeDtypeStruct((N, N), jnp.float32),
)


def kernel(x, edge_index, g0_W1, g0_b1, g0_W2, g0_b2, g1_W1, g1_b1,
           g1_W2, g1_b2, out_W, out_b, ep_W, ep_b):
    zeros = jnp.zeros((N * N,), jnp.float32)
    ones = jnp.ones((128,), jnp.float32)
    a_parts = _make_abuild()(edge_index, zeros, ones)
    a2 = a_parts.reshape(NC, N, N)
    out = _dense(a2, x, g0_W1, g0_b1.reshape(1, D), g0_W2, g0_b2.reshape(1, D),
                 g1_W1, g1_b1.reshape(1, D), g1_W2, g1_b2.reshape(1, D),
                 out_W, out_b.reshape(1, D), ep_W[:D, :],
                 ep_W[D:, :].reshape(1, D), ep_b.reshape(1, 1))
    return out


# speedup vs baseline: 19.0792x; 1.3894x over previous
import functools

import jax
import jax.numpy as jnp
from jax import lax
from jax.experimental import pallas as pl
from jax.experimental.pallas import tpu as pltpu
from jax.experimental.pallas import tpu_sc as plsc

N = 1024
E = 32768
D = 128
NC = 2
NS = 16
NW = NC * NS
EPT = E // NW
NCH = EPT // 16
NB = EPT // 128
ZW = (N * N) // NS
ZBUF = 8192
RPT = N // NS


def _abuild_body(edge_hbm, out_hbm, srcdst_v, idx_v, ones_v, zbuf_v, acc_sh,
                 sem_e, sem_z, sem_s, sem_o):
    c = lax.axis_index("c")
    s = lax.axis_index("s")
    base = (c * NS + s) * EPT
    ec0 = pltpu.async_copy(edge_hbm.at[0, pl.ds(base, EPT)], srcdst_v.at[0], sem_e)
    ec1 = pltpu.async_copy(edge_hbm.at[1, pl.ds(base, EPT)], srcdst_v.at[1], sem_e)

    z16 = jnp.zeros((16,), jnp.float32)

    def _fill_z(i, carry):
        zbuf_v[pl.ds(i * 16, 16)] = z16
        return carry

    lax.fori_loop(0, ZBUF // 16, _fill_z, 0)
    for i in range(8):
        ones_v[pl.ds(i * 16, 16)] = jnp.ones((16,), jnp.float32)

    zcs = [pltpu.async_copy(zbuf_v, acc_sh.at[pl.ds(s * ZW + k * ZBUF, ZBUF)], sem_z)
           for k in range(ZW // ZBUF)]

    ec0.wait()
    ec1.wait()
    for i in range(NCH):
        sv = srcdst_v[0, pl.ds(i * 16, 16)]
        dv = srcdst_v[1, pl.ds(i * 16, 16)]
        idx_v[i // 8, pl.ds((i % 8) * 16, 16)] = dv * N + sv
    for cp in zcs:
        cp.wait()
    plsc.subcore_barrier()
    scs = [pltpu.async_copy(ones_v, acc_sh.at[idx_v.at[b]], sem_s, add=True)
           for b in range(NB)]
    for cp in scs:
        cp.wait()
    plsc.subcore_barrier()
    ocs = []
    for k in range(RPT):
        r = s * RPT + k
        ocs.append(pltpu.async_copy(acc_sh.at[pl.ds(r * N, N)],
                                    out_hbm.at[c, r], sem_o))
    for cp in ocs:
        cp.wait()


@functools.lru_cache(maxsize=None)
def _make_abuild():
    return pl.kernel(
        _abuild_body,
        out_type=jax.ShapeDtypeStruct((NC, N, N), jnp.float32),
        mesh=plsc.VectorSubcoreMesh(core_axis_name="c", subcore_axis_name="s",
                                    num_cores=NC, num_subcores=NS),
        scratch_types=[
            pltpu.VMEM((2, EPT), jnp.int32),
            pltpu.VMEM((NB, 128), jnp.int32),
            pltpu.VMEM((128,), jnp.float32),
            pltpu.VMEM((ZBUF,), jnp.float32),
            pltpu.VMEM_SHARED((N * N,), jnp.float32),
            pltpu.SemaphoreType.DMA,
            pltpu.SemaphoreType.DMA,
            pltpu.SemaphoreType.DMA,
            pltpu.SemaphoreType.DMA,
        ],
    )


def _exact_aggmm(A_bf, h):
    h1 = h.astype(jnp.bfloat16)
    r1 = h - h1.astype(jnp.float32)
    h2 = r1.astype(jnp.bfloat16)
    h3 = (r1 - h2.astype(jnp.float32)).astype(jnp.bfloat16)
    out = jnp.dot(A_bf, h1, preferred_element_type=jnp.float32)
    out += jnp.dot(A_bf, h2, preferred_element_type=jnp.float32)
    out += jnp.dot(A_bf, h3, preferred_element_type=jnp.float32)
    return out


def _dense_body(a_ref, x_ref, w01_ref, b01_ref, w02_ref, b02_ref,
                w11_ref, b11_ref, w12_ref, b12_ref,
                ow_ref, ob_ref, epw1_ref, epw2t_ref, epb_ref, o_ref):
    A_bf = (a_ref[0] + a_ref[1]).astype(jnp.bfloat16)
    x = x_ref[...]
    z = x + _exact_aggmm(A_bf, x)
    t = jnp.maximum(jnp.dot(z, w01_ref[...], preferred_element_type=jnp.float32)
                    + b01_ref[...], 0.0)
    h1 = jnp.maximum(jnp.dot(t, w02_ref[...], preferred_element_type=jnp.float32)
                     + b02_ref[...], 0.0)
    z2 = h1 + _exact_aggmm(A_bf, h1)
    t2 = jnp.maximum(jnp.dot(z2, w11_ref[...], preferred_element_type=jnp.float32)
                     + b11_ref[...], 0.0)
    h2 = jnp.maximum(jnp.dot(t2, w12_ref[...], preferred_element_type=jnp.float32)
                     + b12_ref[...], 0.0)
    hh = jnp.dot(h2, ow_ref[...], preferred_element_type=jnp.float32) + ob_ref[...]
    a = jnp.dot(hh, epw1_ref[...], preferred_element_type=jnp.float32)
    brow = lax.dot_general(epw2t_ref[...], hh, (((1,), (1,)), ((), ())),
                           preferred_element_type=jnp.float32)
    o_ref[...] = a + brow + epb_ref[...]


_dense = pl.pallas_call(
    _dense_body,
    out_shape=jax.ShapeDtypeStruct((N, N), jnp.float32),
)


def kernel(x, edge_index, g0_W1, g0_b1, g0_W2, g0_b2, g1_W1, g1_b1,
           g1_W2, g1_b2, out_W, out_b, ep_W, ep_b):
    a2 = _make_abuild()(edge_index)
    out = _dense(a2, x, g0_W1, g0_b1.reshape(1, D), g0_W2, g0_b2.reshape(1, D),
                 g1_W1, g1_b1.reshape(1, D), g1_W2, g1_b2.reshape(1, D),
                 out_W, out_b.reshape(1, D), ep_W[:D, :],
                 ep_W[D:, :].reshape(1, D), ep_b.reshape(1, 1))
    return out


# speedup vs baseline: 22.3641x; 1.1722x over previous
import functools

import jax
import jax.numpy as jnp
from jax import lax
from jax.experimental import pallas as pl
from jax.experimental.pallas import tpu as pltpu
from jax.experimental.pallas import tpu_sc as plsc

N = 1024
E = 32768
D = 128
NC = 2
NS = 16
HALF = N // NC
EPT = E // NS
NCH = EPT // 16
NB = EPT // 128
ACC = HALF * N + N
ZPT = ACC // NS
ZBUF = ZPT // 4
RPT = HALF // NS


def _abuild_body(edge_hbm, out_hbm, srcdst_v, idx_v, ones_v, zbuf_v, acc_sh,
                 sem_e, sem_z, sem_s, sem_o):
    c = lax.axis_index("c")
    s = lax.axis_index("s")
    base = s * EPT
    ec0 = pltpu.async_copy(edge_hbm.at[0, pl.ds(base, EPT)], srcdst_v.at[0], sem_e)
    ec1 = pltpu.async_copy(edge_hbm.at[1, pl.ds(base, EPT)], srcdst_v.at[1], sem_e)

    z16 = jnp.zeros((16,), jnp.float32)

    def _fill_z(i, carry):
        zbuf_v[pl.ds(i * 16, 16)] = z16
        return carry

    lax.fori_loop(0, ZBUF // 16, _fill_z, 0)
    for i in range(8):
        ones_v[pl.ds(i * 16, 16)] = jnp.ones((16,), jnp.float32)

    zcs = [pltpu.async_copy(zbuf_v, acc_sh.at[pl.ds(s * ZPT + k * ZBUF, ZBUF)], sem_z)
           for k in range(ZPT // ZBUF)]

    ec0.wait()
    ec1.wait()

    row0 = c * HALF

    def _idx(i, carry):
        sv = srcdst_v[0, pl.ds(i * 16, 16)]
        dv = srcdst_v[1, pl.ds(i * 16, 16)]
        local = dv - row0
        ok = (local >= 0) & (local < HALF)
        flat = jnp.where(ok, local * N + sv, HALF * N + sv)
        idx_v[lax.shift_right_logical(i, 3), pl.ds((i & 7) * 16, 16)] = flat
        return carry

    lax.fori_loop(0, NCH, _idx, 0)
    for cp in zcs:
        cp.wait()
    plsc.subcore_barrier()
    scs = [pltpu.async_copy(ones_v, acc_sh.at[idx_v.at[b]], sem_s, add=True)
           for b in range(NB)]
    for cp in scs:
        cp.wait()
    plsc.subcore_barrier()

    def _fire(k, carry):
        r = s * RPT + k
        pltpu.async_copy(acc_sh.at[pl.ds(r * N, N)], out_hbm.at[row0 + r], sem_o)
        return carry

    lax.fori_loop(0, RPT, _fire, 0)

    def _drain(k, carry):
        pltpu.make_async_copy(acc_sh.at[pl.ds(0, N)], out_hbm.at[row0], sem_o).wait()
        return carry

    lax.fori_loop(0, RPT, _drain, 0)


@functools.lru_cache(maxsize=None)
def _make_abuild():
    return pl.kernel(
        _abuild_body,
        out_type=jax.ShapeDtypeStruct((N, N), jnp.float32),
        mesh=plsc.VectorSubcoreMesh(core_axis_name="c", subcore_axis_name="s",
                                    num_cores=NC, num_subcores=NS),
        scratch_types=[
            pltpu.VMEM((2, EPT), jnp.int32),
            pltpu.VMEM((NB, 128), jnp.int32),
            pltpu.VMEM((128,), jnp.float32),
            pltpu.VMEM((ZBUF,), jnp.float32),
            pltpu.VMEM_SHARED((ACC,), jnp.float32),
            pltpu.SemaphoreType.DMA,
            pltpu.SemaphoreType.DMA,
            pltpu.SemaphoreType.DMA,
            pltpu.SemaphoreType.DMA,
        ],
    )


def _exact_aggmm(A_bf, h):
    h1 = h.astype(jnp.bfloat16)
    r1 = h - h1.astype(jnp.float32)
    h2 = r1.astype(jnp.bfloat16)
    h3 = (r1 - h2.astype(jnp.float32)).astype(jnp.bfloat16)
    out = jnp.dot(A_bf, h1, preferred_element_type=jnp.float32)
    out += jnp.dot(A_bf, h2, preferred_element_type=jnp.float32)
    out += jnp.dot(A_bf, h3, preferred_element_type=jnp.float32)
    return out


def _dense_body(a_ref, x_ref, w01_ref, b01_ref, w02_ref, b02_ref,
                w11_ref, b11_ref, w12_ref, b12_ref,
                ow_ref, ob_ref, epw_ref, epb_ref, o_ref):
    A_bf = a_ref[...].astype(jnp.bfloat16)
    x = x_ref[...]
    z = x + _exact_aggmm(A_bf, x)
    t = jnp.maximum(jnp.dot(z, w01_ref[...], preferred_element_type=jnp.float32)
                    + b01_ref[...], 0.0)
    h1 = jnp.maximum(jnp.dot(t, w02_ref[...], preferred_element_type=jnp.float32)
                     + b02_ref[...], 0.0)
    z2 = h1 + _exact_aggmm(A_bf, h1)
    t2 = jnp.maximum(jnp.dot(z2, w11_ref[...], preferred_element_type=jnp.float32)
                     + b11_ref[...], 0.0)
    h2 = jnp.maximum(jnp.dot(t2, w12_ref[...], preferred_element_type=jnp.float32)
                     + b12_ref[...], 0.0)
    hh = jnp.dot(h2, ow_ref[...], preferred_element_type=jnp.float32) + ob_ref[...]
    epw = epw_ref[...]
    a = jnp.dot(hh, epw[:D, :], preferred_element_type=jnp.float32)
    brow = lax.dot_general(epw[D:, :], hh, (((0,), (1,)), ((), ())),
                           preferred_element_type=jnp.float32)
    o_ref[...] = a + brow + epb_ref[...]


_dense = pl.pallas_call(
    _dense_body,
    out_shape=jax.ShapeDtypeStruct((N, N), jnp.float32),
)


def kernel(x, edge_index, g0_W1, g0_b1, g0_W2, g0_b2, g1_W1, g1_b1,
           g1_W2, g1_b2, out_W, out_b, ep_W, ep_b):
    a2 = _make_abuild()(edge_index)
    out = _dense(a2, x, g0_W1, g0_b1, g0_W2, g0_b2,
                 g1_W1, g1_b1, g1_W2, g1_b2,
                 out_W, out_b, ep_W, ep_b)
    return out
